# initial kernel scaffold (unmeasured)
import jax
import jax.numpy as jnp
from jax import lax
from jax.experimental import pallas as pl
from jax.experimental.pallas import tpu as pltpu


def kernel(x, pi):
    def body(x_ref, pi_ref, out_ref, send_sem, recv_sem, copy_sem):
        my_x = lax.axis_index("x")
        my_y = lax.axis_index("y")
        my_z = lax.axis_index("z")
        dst_y = pi_ref[my_y]

        @pl.when(dst_y == my_y)
        def _():
            copy = pltpu.make_async_copy(x_ref, out_ref, copy_sem)
            copy.start()
            copy.wait()

        @pl.when(dst_y != my_y)
        def _():
            rdma = pltpu.make_async_remote_copy(
                src_ref=x_ref,
                dst_ref=out_ref,
                send_sem=send_sem,
                recv_sem=recv_sem,
                device_id=(my_x, dst_y, my_z),
                device_id_type=pl.DeviceIdType.MESH,
            )
            rdma.start()
            rdma.wait()

    return pl.pallas_call(
        body,
        out_shape=jax.ShapeDtypeStruct(x.shape, x.dtype),
        in_specs=[
            pl.BlockSpec(memory_space=pltpu.ANY),
            pl.BlockSpec(memory_space=pltpu.SMEM),
        ],
        out_specs=pl.BlockSpec(memory_space=pltpu.ANY),
        scratch_shapes=[
            pltpu.SemaphoreType.DMA,
            pltpu.SemaphoreType.DMA,
            pltpu.SemaphoreType.DMA,
        ],
    )(x, pi)


# baseline (device time: 394028 ns/iter reference)
import jax
import jax.numpy as jnp
from jax import lax
from jax.experimental import pallas as pl
from jax.experimental.pallas import tpu as pltpu


def kernel(x, pi):
    def body(x_ref, pi_ref, out_ref, send_sem, recv_sem, copy_sem):
        my_x = lax.axis_index("x")
        my_y = lax.axis_index("y")
        my_z = lax.axis_index("z")
        dst_y = pi_ref[my_y]

        @pl.when(dst_y == my_y)
        def _():
            copy = pltpu.make_async_copy(x_ref, out_ref, copy_sem)
            copy.start()
            copy.wait()

        @pl.when(dst_y != my_y)
        def _():
            rdma = pltpu.make_async_remote_copy(
                src_ref=x_ref,
                dst_ref=out_ref,
                send_sem=send_sem,
                recv_sem=recv_sem,
                device_id=(my_x, dst_y, my_z),
                device_id_type=pl.DeviceIdType.MESH,
            )
            rdma.start()
            rdma.wait()

    return pl.pallas_call(
        body,
        out_shape=jax.ShapeDtypeStruct(x.shape, x.dtype),
        in_specs=[
            pl.BlockSpec(memory_space=pl.ANY),
            pl.BlockSpec(memory_space=pltpu.SMEM),
        ],
        out_specs=pl.BlockSpec(memory_space=pl.ANY),
        scratch_shapes=[
            pltpu.SemaphoreType.DMA,
            pltpu.SemaphoreType.DMA,
            pltpu.SemaphoreType.DMA,
        ],
    )(x, pi)


# device time: 387566 ns/iter; 1.0167x vs baseline; 1.0167x over previous
import jax
import jax.numpy as jnp
from jax import lax
from jax.experimental import pallas as pl
from jax.experimental.pallas import tpu as pltpu


def kernel(x, pi):
    def body(x_ref, pi_ref, out_ref, send_sem, recv_sem, copy_sem):
        my_x = lax.axis_index("x")
        my_y = lax.axis_index("y")
        my_z = lax.axis_index("z")
        dst_y = pi_ref[my_y]

        @pl.when(dst_y == my_y)
        def _():
            copy = pltpu.make_async_copy(x_ref, out_ref, copy_sem)
            copy.start()
            copy.wait()

        @pl.when(dst_y != my_y)
        def _():
            barrier_sem = pltpu.get_barrier_semaphore()
            pl.semaphore_signal(
                barrier_sem,
                inc=1,
                device_id=(my_x, dst_y, my_z),
                device_id_type=pl.DeviceIdType.MESH,
            )
            pl.semaphore_wait(barrier_sem, 1)
            rdma = pltpu.make_async_remote_copy(
                src_ref=x_ref,
                dst_ref=out_ref,
                send_sem=send_sem,
                recv_sem=recv_sem,
                device_id=(my_x, dst_y, my_z),
                device_id_type=pl.DeviceIdType.MESH,
            )
            rdma.start()
            rdma.wait()

    return pl.pallas_call(
        body,
        out_shape=jax.ShapeDtypeStruct(x.shape, x.dtype),
        in_specs=[
            pl.BlockSpec(memory_space=pl.ANY),
            pl.BlockSpec(memory_space=pltpu.SMEM),
        ],
        out_specs=pl.BlockSpec(memory_space=pl.ANY),
        scratch_shapes=[
            pltpu.SemaphoreType.DMA,
            pltpu.SemaphoreType.DMA,
            pltpu.SemaphoreType.DMA,
        ],
        compiler_params=pltpu.CompilerParams(collective_id=0),
    )(x, pi)


# device time: 387548 ns/iter; 1.0167x vs baseline; 1.0000x over previous
import jax
import jax.numpy as jnp
from jax import lax
from jax.experimental import pallas as pl
from jax.experimental.pallas import tpu as pltpu


def kernel(x, pi):
    def body(x_ref, pi_ref, out_ref, send_sem, recv_sem, copy_sem):
        my_x = lax.axis_index("x")
        my_y = lax.axis_index("y")
        my_z = lax.axis_index("z")
        dst_y = pi_ref[my_y]

        @pl.when(dst_y == my_y)
        def _():
            copy = pltpu.make_async_copy(x_ref, out_ref, copy_sem)
            copy.start()
            copy.wait()

        @pl.when(dst_y != my_y)
        def _():
            barrier_sem = pltpu.get_barrier_semaphore()
            pl.semaphore_signal(
                barrier_sem,
                inc=1,
                device_id=(my_x, dst_y, my_z),
                device_id_type=pl.DeviceIdType.MESH,
            )
            pl.semaphore_wait(barrier_sem, 1)
            half = x.shape[1] // 2
            rdmas = []
            for c in range(2):
                sl = pl.ds(c * half, half)
                rdmas.append(
                    pltpu.make_async_remote_copy(
                        src_ref=x_ref.at[:, sl],
                        dst_ref=out_ref.at[:, sl],
                        send_sem=send_sem.at[c],
                        recv_sem=recv_sem.at[c],
                        device_id=(my_x, dst_y, my_z),
                        device_id_type=pl.DeviceIdType.MESH,
                    )
                )
            for r in rdmas:
                r.start()
            for r in rdmas:
                r.wait()

    return pl.pallas_call(
        body,
        out_shape=jax.ShapeDtypeStruct(x.shape, x.dtype),
        in_specs=[
            pl.BlockSpec(memory_space=pl.ANY),
            pl.BlockSpec(memory_space=pltpu.SMEM),
        ],
        out_specs=pl.BlockSpec(memory_space=pl.ANY),
        scratch_shapes=[
            pltpu.SemaphoreType.DMA((2,)),
            pltpu.SemaphoreType.DMA((2,)),
            pltpu.SemaphoreType.DMA,
        ],
        compiler_params=pltpu.CompilerParams(collective_id=0),
    )(x, pi)
